# baseline (speedup 1.0000x reference)
import functools

import jax
import jax.numpy as jnp
from jax import lax
from jax.experimental import pallas as pl
from jax.experimental.pallas import tpu as pltpu
from jax.experimental.pallas import tpu_sc as plsc

_BN = 1024
_LANES = 256
_NC = 2
_NS = 16
_L = 16


def _tc_body(x1_ref, x2_ref, x3_ref, x4_ref,
             wm_ref, wmv_ref, wt_ref, wp_ref, wv_ref,
             accp_ref, acch_ref, wscr, *, ea, e, a, d):
    @pl.when(pl.program_id(0) == 0)
    def _pack():
        for j in range(e):
            wscr[:, j * a:(j + 1) * a] = wp_ref[j]
        wscr[:, ea:ea + e] = wm_ref[...]
        wscr[:, ea + e:ea + e + 1] = wmv_ref[...]
        wscr[:, ea + e + 1:ea + 2 * e + 1] = wt_ref[...]
        wscr[:, ea + 2 * e + 1:ea + 3 * e + 1] = wv_ref[..., 0].T
        wscr[:, ea + 3 * e + 1:] = jnp.zeros(
            (d, _LANES - (ea + 3 * e + 1)), jnp.float32)

    dh = x1_ref.shape[1]
    acc = (jnp.dot(x1_ref[...], wscr[:dh], preferred_element_type=jnp.float32)
           + jnp.dot(x2_ref[...], wscr[dh:2 * dh], preferred_element_type=jnp.float32)
           + jnp.dot(x3_ref[...], wscr[2 * dh:3 * dh], preferred_element_type=jnp.float32)
           + jnp.dot(x4_ref[...], wscr[3 * dh:], preferred_element_type=jnp.float32))
    nc = 3 * e + 1
    accp_ref[...] = acc[:, :ea]
    acch_ref[0:nc] = acc[:, ea:ea + nc].T


def _log_pos(x):
    bits = lax.bitcast_convert_type(x, jnp.int32)
    ex = (bits >> 23) - 127
    m = lax.bitcast_convert_type(
        (bits & 0x7FFFFF) | 0x3F800000, jnp.float32)
    z = (m - 1.0) / (m + 1.0)
    z2 = z * z
    ln_m = 2.0 * z * (1.0 + z2 * (1.0 / 3.0 + z2 * (0.2 + z2 * (1.0 / 7.0))))
    return ex.astype(jnp.float32) * 0.6931471805599453 + ln_m


def _sc_body(accp_hbm, acch_hbm, eo_hbm, ft_hbm,
             act_o, val_o, lp_o, no_o, mv_o, mlp_o, tp_o,
             accp_v, acch_v, eo_v, ft_v,
             act_v, val_v, lp_v, no_v, mv_v, mlp_v, tp_v, sem,
             *, e, a, nt):
    wid = lax.axis_index("s") * _NC + lax.axis_index("c")
    base = wid * nt
    sl_tok = pl.ds(base, nt)
    cps = [
        pltpu.make_async_copy(accp_hbm.at[sl_tok, :], accp_v, sem),
        pltpu.make_async_copy(acch_hbm.at[:, sl_tok], acch_v, sem),
        pltpu.make_async_copy(eo_hbm.at[sl_tok], eo_v, sem),
        pltpu.make_async_copy(ft_hbm.at[sl_tok], ft_v, sem),
    ]
    for cp in cps:
        cp.start()
    for cp in cps:
        cp.wait()

    iota = lax.iota(jnp.int32, _L)
    for g in range(nt // _L):
        sl = pl.ds(g * _L, _L)
        cols = iota + (g * _L)

        m0 = acch_v[0, sl]
        mmax = m0
        marg = jnp.zeros((_L,), jnp.int32)
        ms = [m0]
        for f in range(1, e):
            mf = acch_v[f, sl]
            ms.append(mf)
            gt = mf > mmax
            marg = jnp.where(gt, f, marg)
            mmax = jnp.where(gt, mf, mmax)
        msum = jnp.zeros((_L,), jnp.float32)
        for mf in ms:
            msum = msum + jnp.exp(mf - mmax)
        mlp = -_log_pos(msum)
        mval = acch_v[e, sl]

        eo_g = eo_v[sl]
        ft_g = ft_v[sl]
        tlog = plsc.load_gather(acch_v, [eo_g + (e + 1), cols])
        tprob = 1.0 / (1.0 + jnp.exp(-tlog))
        req = (tprob > 0.5) | (ft_g > 0)
        newopt = jnp.where(req, marg, eo_g)
        tout = jnp.where(ft_g > 0, jnp.float32(0.0), tprob)
        val = plsc.load_gather(acch_v, [newopt + (2 * e + 1), cols])

        cbase = newopt * a
        s0 = plsc.load_gather(accp_v, [cols, cbase])
        smax = s0
        sarg = jnp.zeros((_L,), jnp.int32)
        ss = [s0]
        for j in range(1, a):
            sj = plsc.load_gather(accp_v, [cols, cbase + j])
            ss.append(sj)
            gt = sj > smax
            sarg = jnp.where(gt, j, sarg)
            smax = jnp.where(gt, sj, smax)
        ssum = jnp.zeros((_L,), jnp.float32)
        for sj in ss:
            ssum = ssum + jnp.exp(sj - smax)
        lp = -_log_pos(ssum)

        act_v[sl] = sarg
        val_v[sl] = val
        lp_v[sl] = lp
        no_v[sl] = newopt
        mv_v[sl] = mval
        mlp_v[sl] = mlp
        tp_v[sl] = tout

    ocps = [
        pltpu.make_async_copy(act_v, act_o.at[sl_tok], sem),
        pltpu.make_async_copy(val_v, val_o.at[sl_tok], sem),
        pltpu.make_async_copy(lp_v, lp_o.at[sl_tok], sem),
        pltpu.make_async_copy(no_v, no_o.at[sl_tok], sem),
        pltpu.make_async_copy(mv_v, mv_o.at[sl_tok], sem),
        pltpu.make_async_copy(mlp_v, mlp_o.at[sl_tok], sem),
        pltpu.make_async_copy(tp_v, tp_o.at[sl_tok], sem),
    ]
    for cp in ocps:
        cp.start()
    for cp in ocps:
        cp.wait()


def kernel(observation, first_transition, executing_option, Wm, Wmv, Wt, Wp, Wv):
    n, d = observation.shape
    e = Wm.shape[1]
    a = Wp.shape[2]
    ea = e * a
    nh = 32
    nblk = n // _BN
    nt = n // (_NC * _NS)

    eo1 = executing_option.astype(jnp.int32)
    ft1 = first_transition.astype(jnp.int32)

    accp, acch = pl.pallas_call(
        functools.partial(_tc_body, ea=ea, e=e, a=a, d=d),
        grid=(nblk,),
        in_specs=[
            pl.BlockSpec((_BN, d // 4), lambda i: (i, 0)),
            pl.BlockSpec((_BN, d // 4), lambda i: (i, 1)),
            pl.BlockSpec((_BN, d // 4), lambda i: (i, 2)),
            pl.BlockSpec((_BN, d // 4), lambda i: (i, 3)),
            pl.BlockSpec((d, e), lambda i: (0, 0)),
            pl.BlockSpec((d, 1), lambda i: (0, 0)),
            pl.BlockSpec((d, e), lambda i: (0, 0)),
            pl.BlockSpec((e, d, a), lambda i: (0, 0, 0)),
            pl.BlockSpec((e, d, 1), lambda i: (0, 0, 0)),
        ],
        out_specs=[
            pl.BlockSpec((_BN, ea), lambda i: (i, 0)),
            pl.BlockSpec((nh, _BN), lambda i: (0, i)),
        ],
        out_shape=[
            jax.ShapeDtypeStruct((n, ea), jnp.float32),
            jax.ShapeDtypeStruct((nh, n), jnp.float32),
        ],
        scratch_shapes=[pltpu.VMEM((d, _LANES), jnp.float32)],
        compiler_params=pltpu.CompilerParams(
            dimension_semantics=("arbitrary",)),
    )(observation, observation, observation, observation, Wm, Wmv, Wt, Wp, Wv)

    f32, i32 = jnp.float32, jnp.int32
    sc = pl.kernel(
        functools.partial(_sc_body, e=e, a=a, nt=nt),
        mesh=plsc.VectorSubcoreMesh(core_axis_name="c", subcore_axis_name="s"),
        compiler_params=pltpu.CompilerParams(needs_layout_passes=False),
        out_type=[
            jax.ShapeDtypeStruct((n,), i32),
            jax.ShapeDtypeStruct((n,), f32),
            jax.ShapeDtypeStruct((n,), f32),
            jax.ShapeDtypeStruct((n,), i32),
            jax.ShapeDtypeStruct((n,), f32),
            jax.ShapeDtypeStruct((n,), f32),
            jax.ShapeDtypeStruct((n,), f32),
        ],
        scratch_types=[
            pltpu.VMEM((nt, ea), f32),
            pltpu.VMEM((nh, nt), f32),
            pltpu.VMEM((nt,), i32),
            pltpu.VMEM((nt,), i32),
            pltpu.VMEM((nt,), i32),
            pltpu.VMEM((nt,), f32),
            pltpu.VMEM((nt,), f32),
            pltpu.VMEM((nt,), i32),
            pltpu.VMEM((nt,), f32),
            pltpu.VMEM((nt,), f32),
            pltpu.VMEM((nt,), f32),
            pltpu.SemaphoreType.DMA,
        ],
    )
    return tuple(sc(accp, acch, eo1, ft1))


# speedup vs baseline: 1.2180x; 1.2180x over previous
import functools

import jax
import jax.numpy as jnp
from jax import lax
from jax.experimental import pallas as pl
from jax.experimental.pallas import tpu as pltpu
from jax.experimental.pallas import tpu_sc as plsc

_BN = 1024
_NC = 2
_NS = 16
_L = 16


def _tc_body(x1_ref, x2_ref, x3_ref, x4_ref, wp_ref, wh_ref,
             accp_ref, acch_ref, *, nc):
    dh = x1_ref.shape[1]
    xs = (x1_ref, x2_ref, x3_ref, x4_ref)
    accp = None
    acch = None
    for k, x_ref in enumerate(xs):
        x = x_ref[...]
        pp = jnp.dot(x, wp_ref[pl.ds(k * dh, dh), :],
                     preferred_element_type=jnp.float32)
        hh = jnp.dot(x, wh_ref[pl.ds(k * dh, dh), :],
                     preferred_element_type=jnp.float32)
        accp = pp if accp is None else accp + pp
        acch = hh if acch is None else acch + hh
    accp_ref[...] = accp
    acch_ref[0:nc] = acch.T


def _log_pos(x):
    bits = lax.bitcast_convert_type(x, jnp.int32)
    ex = (bits >> 23) - 127
    m = lax.bitcast_convert_type(
        (bits & 0x7FFFFF) | 0x3F800000, jnp.float32)
    z = (m - 1.0) / (m + 1.0)
    z2 = z * z
    ln_m = 2.0 * z * (1.0 + z2 * (1.0 / 3.0 + z2 * (0.2 + z2 * (1.0 / 7.0))))
    return ex.astype(jnp.float32) * 0.6931471805599453 + ln_m


def _sc_body(accp_hbm, acch_hbm, eo_hbm, ft_hbm,
             act_o, val_o, lp_o, no_o, mv_o, mlp_o, tp_o,
             accp_v, acch_v, eo_v, ft_v,
             act_v, val_v, lp_v, no_v, mv_v, mlp_v, tp_v, sem,
             *, e, a, nt):
    wid = lax.axis_index("s") * _NC + lax.axis_index("c")
    base = wid * nt
    sl_tok = pl.ds(base, nt)
    cps = [
        pltpu.make_async_copy(accp_hbm.at[sl_tok, :], accp_v, sem),
        pltpu.make_async_copy(acch_hbm.at[:, sl_tok], acch_v, sem),
        pltpu.make_async_copy(eo_hbm.at[sl_tok], eo_v, sem),
        pltpu.make_async_copy(ft_hbm.at[sl_tok], ft_v, sem),
    ]
    for cp in cps:
        cp.start()
    for cp in cps:
        cp.wait()

    iota = lax.iota(jnp.int32, _L)
    for g in range(nt // _L):
        sl = pl.ds(g * _L, _L)
        cols = iota + (g * _L)

        m0 = acch_v[0, sl]
        mmax = m0
        marg = jnp.zeros((_L,), jnp.int32)
        ms = [m0]
        for f in range(1, e):
            mf = acch_v[f, sl]
            ms.append(mf)
            gt = mf > mmax
            marg = jnp.where(gt, f, marg)
            mmax = jnp.where(gt, mf, mmax)
        msum = jnp.zeros((_L,), jnp.float32)
        for mf in ms:
            msum = msum + jnp.exp(mf - mmax)
        mlp = -_log_pos(msum)
        mval = acch_v[e, sl]

        eo_g = eo_v[sl]
        ft_g = ft_v[sl]
        tlog = plsc.load_gather(acch_v, [eo_g + (e + 1), cols])
        tprob = 1.0 / (1.0 + jnp.exp(-tlog))
        req = (tprob > 0.5) | (ft_g > 0)
        newopt = jnp.where(req, marg, eo_g)
        tout = jnp.where(ft_g > 0, jnp.float32(0.0), tprob)
        val = plsc.load_gather(acch_v, [newopt + (2 * e + 1), cols])

        cbase = newopt * a
        s0 = plsc.load_gather(accp_v, [cols, cbase])
        smax = s0
        sarg = jnp.zeros((_L,), jnp.int32)
        ss = [s0]
        for j in range(1, a):
            sj = plsc.load_gather(accp_v, [cols, cbase + j])
            ss.append(sj)
            gt = sj > smax
            sarg = jnp.where(gt, j, sarg)
            smax = jnp.where(gt, sj, smax)
        ssum = jnp.zeros((_L,), jnp.float32)
        for sj in ss:
            ssum = ssum + jnp.exp(sj - smax)
        lp = -_log_pos(ssum)

        act_v[sl] = sarg
        val_v[sl] = val
        lp_v[sl] = lp
        no_v[sl] = newopt
        mv_v[sl] = mval
        mlp_v[sl] = mlp
        tp_v[sl] = tout

    ocps = [
        pltpu.make_async_copy(act_v, act_o.at[sl_tok], sem),
        pltpu.make_async_copy(val_v, val_o.at[sl_tok], sem),
        pltpu.make_async_copy(lp_v, lp_o.at[sl_tok], sem),
        pltpu.make_async_copy(no_v, no_o.at[sl_tok], sem),
        pltpu.make_async_copy(mv_v, mv_o.at[sl_tok], sem),
        pltpu.make_async_copy(mlp_v, mlp_o.at[sl_tok], sem),
        pltpu.make_async_copy(tp_v, tp_o.at[sl_tok], sem),
    ]
    for cp in ocps:
        cp.start()
    for cp in ocps:
        cp.wait()


def kernel(observation, first_transition, executing_option, Wm, Wmv, Wt, Wp, Wv):
    n, d = observation.shape
    e = Wm.shape[1]
    a = Wp.shape[2]
    ea = e * a
    nc = 3 * e + 1
    nh = 32
    nblk = n // _BN
    nt = n // (_NC * _NS)

    wp_flat = jnp.transpose(Wp, (1, 0, 2)).reshape(d, ea)
    whead = jnp.concatenate([Wm, Wmv, Wt, Wv[..., 0].T], axis=1)
    eo1 = executing_option.astype(jnp.int32)
    ft1 = first_transition.astype(jnp.int32)

    accp, acch = pl.pallas_call(
        functools.partial(_tc_body, nc=nc),
        grid=(nblk,),
        in_specs=[
            pl.BlockSpec((_BN, d // 4), lambda i: (i, 0)),
            pl.BlockSpec((_BN, d // 4), lambda i: (i, 1)),
            pl.BlockSpec((_BN, d // 4), lambda i: (i, 2)),
            pl.BlockSpec((_BN, d // 4), lambda i: (i, 3)),
            pl.BlockSpec((d, ea), lambda i: (0, 0)),
            pl.BlockSpec((d, nc), lambda i: (0, 0)),
        ],
        out_specs=[
            pl.BlockSpec((_BN, ea), lambda i: (i, 0)),
            pl.BlockSpec((nh, _BN), lambda i: (0, i)),
        ],
        out_shape=[
            jax.ShapeDtypeStruct((n, ea), jnp.float32),
            jax.ShapeDtypeStruct((nh, n), jnp.float32),
        ],
        compiler_params=pltpu.CompilerParams(
            dimension_semantics=("arbitrary",)),
    )(observation, observation, observation, observation, wp_flat, whead)

    f32, i32 = jnp.float32, jnp.int32
    sc = pl.kernel(
        functools.partial(_sc_body, e=e, a=a, nt=nt),
        mesh=plsc.VectorSubcoreMesh(core_axis_name="c", subcore_axis_name="s"),
        compiler_params=pltpu.CompilerParams(needs_layout_passes=False),
        out_type=[
            jax.ShapeDtypeStruct((n,), i32),
            jax.ShapeDtypeStruct((n,), f32),
            jax.ShapeDtypeStruct((n,), f32),
            jax.ShapeDtypeStruct((n,), i32),
            jax.ShapeDtypeStruct((n,), f32),
            jax.ShapeDtypeStruct((n,), f32),
            jax.ShapeDtypeStruct((n,), f32),
        ],
        scratch_types=[
            pltpu.VMEM((nt, ea), f32),
            pltpu.VMEM((nh, nt), f32),
            pltpu.VMEM((nt,), i32),
            pltpu.VMEM((nt,), i32),
            pltpu.VMEM((nt,), i32),
            pltpu.VMEM((nt,), f32),
            pltpu.VMEM((nt,), f32),
            pltpu.VMEM((nt,), i32),
            pltpu.VMEM((nt,), f32),
            pltpu.VMEM((nt,), f32),
            pltpu.VMEM((nt,), f32),
            pltpu.SemaphoreType.DMA,
        ],
    )
    return tuple(sc(accp, acch, eo1, ft1))
